# baseline (speedup 1.0000x reference)
import functools
import numpy as np
import jax
import jax.numpy as jnp
from jax import lax
from jax.experimental import pallas as pl
from jax.experimental.pallas import tpu as pltpu
from jax.experimental.pallas import tpu_sc as plsc

N, E, D = 10000, 320000, 128
HEADS = 8
DH = D // HEADS
NB = 1000

NC, NS = 2, 16
NW = NC * NS
NP = 10240
HALF = NP // 2
UROWS = 5248
URPT = UROWS // NS
DUMP = 5200
SUB = 128
CH = 256
EP = 360448
EPT = EP // NS
IRT = EPT // SUB
GR = 4
GROUPS = IRT // GR
CPG = GR * SUB // CH
AW = 144


_PROBE_SKIP_AGATHER = True
_PROBE_SKIP_SCALE = True
_PROBE_SKIP_USCATTER = True
_PROBE_SKIP_HGATHER = True
_PROBE_SKIP_DSCATTER = True


def _leaky(x, slope=0.2):
    return jnp.where(x >= 0, x, slope * x)


_GDN = lax.GatherDimensionNumbers(
    offset_dims=(), collapsed_slice_dims=(0,), start_index_map=(0,))


def _lane_gather(vec, idx):
    return lax.gather(vec, idx[:, None], dimension_numbers=_GDN,
                      slice_sizes=(1,),
                      mode=lax.GatherScatterMode.PROMISE_IN_BOUNDS)


def _prologue_body(x_ref, w_in_ref, b_in_ref, gamma1_ref, beta1_ref,
                   w1_ref, asrc1_ref, adst1_ref,
                   haug_ref, adst_tab_ref):
    x = jnp.nan_to_num(x_ref[...], nan=0.0)
    x_in = jnp.dot(x, w_in_ref[...], preferred_element_type=jnp.float32)
    x_in = x_in + b_in_ref[...]
    h = _leaky(x_in, 0.2)
    h = gamma1_ref[...] * h / jnp.sqrt(1.0 + 1e-5) + beta1_ref[...]
    h1 = jnp.dot(h, w1_ref[...], preferred_element_type=jnp.float32)
    h1r = h1.reshape(NB, HEADS, DH)
    a_src = (h1r * asrc1_ref[...][None]).sum(-1)
    a_dst = (h1r * adst1_ref[...][None]).sum(-1)
    z = jnp.zeros((NB, 8), jnp.float32)
    haug_ref[...] = jnp.concatenate([h1, a_src, a_dst], axis=1)
    adst_tab_ref[...] = jnp.concatenate([a_dst, z], axis=1)


def _prologue(x, W_in, b_in, gamma1, beta1, W1, att_src1, att_dst1):
    return pl.pallas_call(
        _prologue_body,
        grid=(N // NB,),
        in_specs=[
            pl.BlockSpec((NB, D), lambda i: (i, 0)),
            pl.BlockSpec((D, D), lambda i: (0, 0)),
            pl.BlockSpec((D,), lambda i: (0,)),
            pl.BlockSpec((D,), lambda i: (0,)),
            pl.BlockSpec((D,), lambda i: (0,)),
            pl.BlockSpec((D, D), lambda i: (0, 0)),
            pl.BlockSpec((HEADS, DH), lambda i: (0, 0)),
            pl.BlockSpec((HEADS, DH), lambda i: (0, 0)),
        ],
        out_specs=[
            pl.BlockSpec((NB, AW), lambda i: (i, 0)),
            pl.BlockSpec((NB, 16), lambda i: (i, 0)),
        ],
        out_shape=[
            jax.ShapeDtypeStruct((N, AW), jnp.float32),
            jax.ShapeDtypeStruct((N, 16), jnp.float32),
        ],
    )(x, W_in, b_in, gamma1, beta1, W1, att_src1, att_dst1)


def _mid_body(u_ref, den_ref, expand_ref, b1_ref, w2_ref, asrc2_ref, adst2_ref,
              hskip_ref, haug_ref, adst_tab_ref):
    u = u_ref[...]
    den8 = den_ref[:, 0:8]
    rden = 1.0 / (den8 + 1e-16)
    scale = jnp.dot(rden, expand_ref[...], preferred_element_type=jnp.float32)
    o1 = u * scale
    h = _leaky(o1 + b1_ref[...], 0.2)
    hskip_ref[...] = h
    h2 = jnp.dot(h, w2_ref[...], preferred_element_type=jnp.float32)
    a2s = (h2 * asrc2_ref[...]).sum(-1, keepdims=True)
    a2d = (h2 * adst2_ref[...]).sum(-1, keepdims=True)
    z7 = jnp.zeros((NB, 7), jnp.float32)
    z15 = jnp.zeros((NB, 15), jnp.float32)
    haug_ref[...] = jnp.concatenate([h2, a2s, z7, a2d, z7], axis=1)
    adst_tab_ref[...] = jnp.concatenate([a2d, z15], axis=1)


def _mid(u1, den1, b1, W2, att_src2, att_dst2):
    expand = jnp.asarray(np.kron(np.eye(8, dtype=np.float32),
                                 np.ones((1, 16), np.float32)))
    return pl.pallas_call(
        _mid_body,
        grid=(N // NB,),
        in_specs=[
            pl.BlockSpec((NB, D), lambda i: (i, 0)),
            pl.BlockSpec((NB, 16), lambda i: (i, 0)),
            pl.BlockSpec((8, D), lambda i: (0, 0)),
            pl.BlockSpec((D,), lambda i: (0,)),
            pl.BlockSpec((D, D), lambda i: (0, 0)),
            pl.BlockSpec((1, D), lambda i: (0, 0)),
            pl.BlockSpec((1, D), lambda i: (0, 0)),
        ],
        out_specs=[
            pl.BlockSpec((NB, D), lambda i: (i, 0)),
            pl.BlockSpec((NB, AW), lambda i: (i, 0)),
            pl.BlockSpec((NB, 16), lambda i: (i, 0)),
        ],
        out_shape=[
            jax.ShapeDtypeStruct((N, D), jnp.float32),
            jax.ShapeDtypeStruct((N, AW), jnp.float32),
            jax.ShapeDtypeStruct((N, 16), jnp.float32),
        ],
    )(u1, den1, expand, b1, W2, att_src2, att_dst2)


def _epilogue_body(u_ref, den_ref, hskip_ref, wskip_ref, b2_ref, bskip_ref,
                   gamma2_ref, beta2_ref, out_ref):
    u = u_ref[...]
    den = den_ref[:, 0:1]
    o2 = u * (1.0 / (den + 1e-16))
    h = o2 + b2_ref[...]
    h = h + jnp.dot(hskip_ref[...], wskip_ref[...],
                    preferred_element_type=jnp.float32) + bskip_ref[...]
    h = gamma2_ref[...] * h / jnp.sqrt(1.0 + 1e-5) + beta2_ref[...]
    h = jnp.nan_to_num(h, nan=0.0)
    norm = jnp.maximum(jnp.sqrt((h * h).sum(-1, keepdims=True)), 1e-12)
    out_ref[...] = h / norm


def _epilogue(u2, den2, h_skip, W_skip, b2, b_skip, gamma2, beta2):
    return pl.pallas_call(
        _epilogue_body,
        grid=(N // NB,),
        in_specs=[
            pl.BlockSpec((NB, D), lambda i: (i, 0)),
            pl.BlockSpec((NB, 16), lambda i: (i, 0)),
            pl.BlockSpec((NB, D), lambda i: (i, 0)),
            pl.BlockSpec((D, D), lambda i: (0, 0)),
            pl.BlockSpec((D,), lambda i: (0,)),
            pl.BlockSpec((D,), lambda i: (0,)),
            pl.BlockSpec((D,), lambda i: (0,)),
            pl.BlockSpec((D,), lambda i: (0,)),
        ],
        out_specs=pl.BlockSpec((NB, D), lambda i: (i, 0)),
        out_shape=jax.ShapeDtypeStruct((N, D), jnp.float32),
    )(u2, den2, h_skip, W_skip, b2, b_skip, gamma2, beta2)


def _edge_body(heads, haug_hbm, adst_hbm, zer_hbm,
               src_hbm, dst_hbm, acc_hbm,
               idx_s, idx_d, idx_m, mbuf, bufD,
               acc_sh, sem_g0, sem_g1, sem_s0, sem_s1):
    cid = lax.axis_index("c")
    sid = lax.axis_index("s")

    ubase = sid * URPT
    for k in range(URPT // 32):
        pltpu.sync_copy(zer_hbm, acc_sh.at[pl.ds(ubase + k * 32, 32)])
    pltpu.sync_copy(zer_hbm.at[pl.ds(0, URPT % 32)],
                    acc_sh.at[pl.ds(ubase + URPT - URPT % 32, URPT % 32)])
    plsc.subcore_barrier()

    bcast_idx = [jnp.full((16,), j, jnp.int32) for j in range(heads)]
    sems_g = [sem_g0, sem_g1]
    sems_s = [sem_s0, sem_s1]
    lo = cid * HALF
    epg = GR * SUB

    @pl.loop(0, GROUPS)
    def _(g):
        gbase = sid * EPT + g * epg
        pltpu.sync_copy(src_hbm.at[pl.ds(gbase, epg)], idx_s)
        pltpu.sync_copy(dst_hbm.at[pl.ds(gbase, epg)], idx_d)

        @pl.loop(0, GR)
        def _(r):
            for q in range(8):
                d = idx_d[pl.ds(r * SUB + q * 16, 16)]
                u = d - lo
                bad = (u < 0) | (u >= HALF)
                idx_m[r, pl.ds(q * 16, 16)] = jnp.where(bad, DUMP, u)

        def issue_gathers(k):
            p = k % 2
            sl = pl.ds(k * CH, CH)
            return [
                pltpu.async_copy(haug_hbm.at[idx_s.at[sl]], mbuf.at[p],
                                 sems_g[p]),
                pltpu.async_copy(adst_hbm.at[idx_d.at[sl]], bufD.at[p],
                                 sems_g[p]),
            ]

        gd = {0: issue_gathers(0)}
        sd = {}
        for k in range(CPG):
            p = k % 2
            if k >= 1:
                for c in sd.pop(k - 1):
                    c.wait()
            if k + 1 < CPG:
                gd[k + 1] = issue_gathers(k + 1)
            for c in gd.pop(k):
                c.wait()

            @pl.loop(0, CH, unroll=4)
            def _(e):
                vs = mbuf[p, e, pl.ds(D, 16)]
                vd = bufD[p, e, :]
                xx = vs + vd
                y = jnp.where(xx >= 0, xx, 0.2 * xx)
                ex = jnp.exp(y)
                mbuf[p, e, pl.ds(D, 16)] = ex
                if heads == 1:
                    b0 = _lane_gather(ex, bcast_idx[0])
                    for j in range(8):
                        sl = pl.ds(j * 16, 16)
                        mbuf[p, e, sl] = mbuf[p, e, sl] * b0
                else:
                    for j in range(8):
                        sl = pl.ds(j * 16, 16)
                        bj = _lane_gather(ex, bcast_idx[j])
                        mbuf[p, e, sl] = mbuf[p, e, sl] * bj

            sd[k] = [
                pltpu.async_copy(mbuf.at[p, pl.ds(0, SUB)],
                                 acc_sh.at[idx_m.at[2 * k]],
                                 sems_s[p], add=True),
                pltpu.async_copy(mbuf.at[p, pl.ds(SUB, SUB)],
                                 acc_sh.at[idx_m.at[2 * k + 1]],
                                 sems_s[p], add=True),
            ]
        for c in sd.pop(CPG - 1):
            c.wait()

    plsc.subcore_barrier()
    pltpu.sync_copy(acc_sh.at[pl.ds(ubase, URPT)],
                    acc_hbm.at[cid].at[pl.ds(ubase, URPT)])


def _edge_pass(heads, haug_tab, adst_tab, src_flat, dst_flat):
    mesh = plsc.VectorSubcoreMesh(core_axis_name="c", subcore_axis_name="s")
    zer = jnp.zeros((32, AW), jnp.float32)
    kern = pl.kernel(
        functools.partial(_edge_body, heads),
        compiler_params=pltpu.CompilerParams(use_tc_tiling_on_sc=False),
        out_type=[
            jax.ShapeDtypeStruct((2, UROWS, AW), jnp.float32),
        ],
        mesh=mesh,
        scratch_types=[
            pltpu.VMEM((GR * SUB,), jnp.int32),
            pltpu.VMEM((GR * SUB,), jnp.int32),
            pltpu.VMEM((GR, SUB), jnp.int32),
            pltpu.VMEM((2, CH, AW), jnp.float32),
            pltpu.VMEM((2, CH, 16), jnp.float32),
            pltpu.VMEM_SHARED((UROWS, AW), jnp.float32),
            pltpu.SemaphoreType.DMA,
            pltpu.SemaphoreType.DMA,
            pltpu.SemaphoreType.DMA,
            pltpu.SemaphoreType.DMA,
        ],
    )
    return kern(haug_tab, adst_tab, zer, src_flat, dst_flat)


def kernel(x, edge_index, W_in, b_in, gamma1, beta1, W1, att_src1, att_dst1, b1,
           W2, att_src2, att_dst2, b2, W_skip, b_skip, gamma2, beta2):
    ar = jnp.arange(N, dtype=jnp.int32)
    pad = jnp.full((EP - E - N,), N, jnp.int32)
    src_flat = jnp.concatenate([edge_index[0].astype(jnp.int32), ar, pad])
    dst_flat = jnp.concatenate([edge_index[1].astype(jnp.int32), ar, pad])

    haug1, adst1 = _prologue(
        x, W_in, b_in, gamma1, beta1, W1, att_src1, att_dst1)
    haug1p = jnp.pad(haug1, ((0, NP - N), (0, 0)))
    adst1p = jnp.pad(adst1, ((0, NP - N), (0, 0)))

    acc1 = _edge_pass(HEADS, haug1p, adst1p, src_flat, dst_flat)[0]
    accf = jnp.concatenate([acc1[0, :HALF], acc1[1, :N - HALF]])
    u1f = accf[:, :D]
    den1f = accf[:, D:]

    h_skip, haug2, adst2 = _mid(
        u1f, den1f, b1, W2, att_src2, att_dst2)
    haug2p = jnp.pad(haug2, ((0, NP - N), (0, 0)))
    adst2p = jnp.pad(adst2, ((0, NP - N), (0, 0)))

    acc2 = _edge_pass(1, haug2p, adst2p, src_flat, dst_flat)[0]
    accf2 = jnp.concatenate([acc2[0, :HALF], acc2[1, :N - HALF]])

    return _epilogue(accf2[:, :D], accf2[:, D:], h_skip, W_skip,
                     b2, b_skip, gamma2, beta2)
